# baseline (speedup 1.0000x reference)
import functools
import math

import jax
import jax.numpy as jnp
from jax import lax
from jax.experimental import pallas as pl
from jax.experimental.pallas import tpu as pltpu
from jax.experimental.pallas import tpu_sc as plsc

L = 16
ROW = 128
CH = 4


def _make_kernel(V, D, NROWS, NW, NC):
    rows_per_w = NROWS // NW
    nchunks = rows_per_w // CH
    vregs_per_row = D // L

    mesh = plsc.VectorSubcoreMesh(core_axis_name="c", subcore_axis_name="s")

    @functools.partial(
        pl.kernel,
        out_type=jax.ShapeDtypeStruct((NROWS, ROW, D), jnp.float32),
        mesh=mesh,
        scratch_types=[
            pltpu.VMEM((CH, ROW), jnp.int32),
            pltpu.VMEM((CH, ROW, D), jnp.float32),
            pltpu.SemaphoreType.DMA,
        ],
        compiler_params=pltpu.CompilerParams(use_tc_tiling_on_sc=False),
    )
    def emb(tbl_hbm, idx_hbm, out_hbm, idx_v, rows_v, gsem):
        wid = lax.axis_index("s") * NC + lax.axis_index("c")
        wbase = wid * rows_per_w

        def chunk(c, carry):
            r0 = wbase + c * CH
            pltpu.sync_copy(idx_hbm.at[pl.ds(r0, CH)], idx_v)
            cps = [
                pltpu.async_copy(tbl_hbm.at[idx_v.at[j]], rows_v.at[j], gsem)
                for j in range(CH)
            ]
            for cp in cps:
                cp.wait()

            @plsc.parallel_loop(0, ROW, unroll=2)
            def _scale(i):
                for j in range(CH):
                    for l in range(vregs_per_row):
                        rows_v[j, i, pl.ds(l * L, L)] = (
                            rows_v[j, i, pl.ds(l * L, L)] * 8.0
                        )

            pltpu.sync_copy(rows_v, out_hbm.at[pl.ds(r0, CH)])
            return carry

        lax.fori_loop(0, nchunks, chunk, 0)

    return emb


def kernel(x, lut):
    B0, B1 = x.shape
    V, D = lut.shape
    B = B0 * B1
    NROWS = B // ROW
    info = plsc.get_sparse_core_info()
    NC, NS = info.num_cores, info.num_subcores
    NW = NC * NS
    idx = x.reshape(NROWS, ROW).astype(jnp.int32)
    out = _make_kernel(V, D, NROWS, NW, NC)(lut, idx)
    return out.reshape(B0, B1, D)


# speedup vs baseline: 1.0180x; 1.0180x over previous
import functools
import math

import jax
import jax.numpy as jnp
from jax import lax
from jax.experimental import pallas as pl
from jax.experimental.pallas import tpu as pltpu
from jax.experimental.pallas import tpu_sc as plsc

L = 16
CHR = 4
SPLITS = ((0, 104), (104, 96))


def _make_kernel(V, D, B0, B1, NW, NC):
    rows_per_w = B0 // NW
    nchunks = rows_per_w // CHR
    vregs_per_row = D // L

    mesh = plsc.VectorSubcoreMesh(core_axis_name="c", subcore_axis_name="s")

    @functools.partial(
        pl.kernel,
        out_type=jax.ShapeDtypeStruct((B0, B1, D), jnp.float32),
        mesh=mesh,
        scratch_types=[
            pltpu.VMEM((CHR, B1), jnp.int32),
            pltpu.VMEM((CHR, B1, D), jnp.float32),
            pltpu.SemaphoreType.DMA,
        ],
        compiler_params=pltpu.CompilerParams(use_tc_tiling_on_sc=False),
    )
    def emb(tbl_hbm, idx_hbm, out_hbm, idx_v, rows_v, gsem):
        wid = lax.axis_index("s") * NC + lax.axis_index("c")
        wbase = wid * rows_per_w

        def chunk(c, carry):
            r0 = wbase + c * CHR
            pltpu.sync_copy(idx_hbm.at[pl.ds(r0, CHR)], idx_v)
            cps = [
                pltpu.async_copy(
                    tbl_hbm.at[idx_v.at[j, pl.ds(o, n)]],
                    rows_v.at[j, pl.ds(o, n)],
                    gsem,
                )
                for j in range(CHR)
                for (o, n) in SPLITS
            ]
            for cp in cps:
                cp.wait()

            @plsc.parallel_loop(0, B1, unroll=2)
            def _scale(i):
                for j in range(CHR):
                    for l in range(vregs_per_row):
                        rows_v[j, i, pl.ds(l * L, L)] = (
                            rows_v[j, i, pl.ds(l * L, L)] * 8.0
                        )

            pltpu.sync_copy(rows_v, out_hbm.at[pl.ds(r0, CHR)])
            return carry

        lax.fori_loop(0, nchunks, chunk, 0)

    return emb


def kernel(x, lut):
    B0, B1 = x.shape
    V, D = lut.shape
    info = plsc.get_sparse_core_info()
    NC, NS = info.num_cores, info.num_subcores
    NW = NC * NS
    return _make_kernel(V, D, B0, B1, NW, NC)(lut, x.astype(jnp.int32))
